# baseline (speedup 1.0000x reference)
import functools

import jax
import jax.numpy as jnp
from jax import lax
from jax.experimental import pallas as pl
from jax.experimental.pallas import tpu as pltpu
from jax.experimental.pallas import tpu_sc as plsc

N_NODES = 10000
N_EDGES = 320000
NFEAT = 128
NHID = 128
NCLASS = 64

NW = 32
N_PAD = 10240
E_PAD = 327680
EPT = E_PAD // NW
C = 128
NCHUNK = EPT // C
ROWS_PER_TILE = N_PAD // 16


def _mm_body(x_ref, w_ref, o_ref):
    o_ref[...] = jnp.dot(x_ref[...], w_ref[...],
                         preferred_element_type=jnp.float32)


def _matmul(x, w, bm=1024):
    m, k = x.shape
    n = w.shape[1]
    return pl.pallas_call(
        _mm_body,
        grid=(m // bm,),
        in_specs=[pl.BlockSpec((bm, k), lambda i: (i, 0)),
                  pl.BlockSpec((k, n), lambda i: (0, 0))],
        out_specs=pl.BlockSpec((bm, n), lambda i: (i, 0)),
        out_shape=jax.ShapeDtypeStruct((m, n), jnp.float32),
    )(x, w)


def _mm2_body(p_ref, w_ref, o_ref):
    h = jnp.maximum(p_ref[0] + p_ref[1], 0.0)
    o_ref[...] = jnp.dot(h, w_ref[...], preferred_element_type=jnp.float32)


def _relu_sum_matmul(p, w, bm=1024):
    _, m, k = p.shape
    n = w.shape[1]
    return pl.pallas_call(
        _mm2_body,
        grid=(m // bm,),
        in_specs=[pl.BlockSpec((2, bm, k), lambda i: (0, i, 0)),
                  pl.BlockSpec((k, n), lambda i: (0, 0))],
        out_specs=pl.BlockSpec((bm, n), lambda i: (i, 0)),
        out_shape=jax.ShapeDtypeStruct((m, n), jnp.float32),
    )(p, w)


def _sum2_body(p_ref, o_ref):
    o_ref[...] = p_ref[0] + p_ref[1]


def _sum2(p, bm=1024):
    _, m, n = p.shape
    return pl.pallas_call(
        _sum2_body,
        grid=(m // bm,),
        in_specs=[pl.BlockSpec((2, bm, n), lambda i: (0, i, 0))],
        out_specs=pl.BlockSpec((bm, n), lambda i: (i, 0)),
        out_shape=jax.ShapeDtypeStruct((m, n), jnp.float32),
    )(p)


_GATHER_DNUMS = lax.GatherDimensionNumbers(
    offset_dims=(), collapsed_slice_dims=(0,), start_index_map=(0,))


def _lane_splat(vec, lane):
    idx = jnp.full((16, 1), lane, jnp.int32)
    return lax.gather(vec, idx, _GATHER_DNUMS, slice_sizes=(1,),
                      mode=lax.GatherScatterMode.PROMISE_IN_BOUNDS)


def _make_agg(F):
    mesh = plsc.VectorSubcoreMesh(core_axis_name="c", subcore_axis_name="s")

    @functools.partial(
        pl.kernel,
        out_type=jax.ShapeDtypeStruct((2, N_PAD, F), jnp.float32),
        mesh=mesh,
        compiler_params=pltpu.CompilerParams(use_tc_tiling_on_sc=False,
                                             needs_layout_passes=False),
        scratch_types=[
            pltpu.VMEM_SHARED((N_PAD, F), jnp.float32),
            pltpu.VMEM((4, 3, C), jnp.int32),
            pltpu.VMEM((C, F), jnp.float32),
            pltpu.VMEM((C, F), jnp.float32),
            pltpu.SemaphoreType.DMA,
            pltpu.SemaphoreType.DMA,
            pltpu.SemaphoreType.DMA,
            pltpu.SemaphoreType.DMA,
            pltpu.SemaphoreType.DMA,
            pltpu.SemaphoreType.DMA,
        ],
    )
    def agg(h_hbm, edges_hbm, out_hbm,
            acc_sh, stage_v, rows0_v, rows1_v,
            gsem0, gsem1, ssem0, ssem1, ssem2, ssem3):
        c = lax.axis_index("c")
        s = lax.axis_index("s")
        wid = s * 2 + (1 - c)
        rows = (rows0_v, rows1_v)
        gsems = (gsem0, gsem1)
        ssems = (ssem0, ssem1, ssem2, ssem3)

        def stage_start(k, sb):
            pltpu.async_copy(edges_hbm.at[wid, k], stage_v.at[sb], ssems[sb])

        def stage_wait(k, sb):
            pltpu.make_async_copy(edges_hbm.at[wid, k], stage_v.at[sb],
                                  ssems[sb]).wait()

        def gather_start(sb, rb):
            pltpu.async_copy(h_hbm.at[stage_v.at[sb, 0]], rows[rb], gsems[rb])

        def gather_wait(sb, rb):
            pltpu.make_async_copy(h_hbm.at[stage_v.at[sb, 0]], rows[rb],
                                  gsems[rb]).wait()

        for sb in range(4):
            stage_start(sb, sb)

        def zrow(r, carry):
            for j in range(F // 16):
                rows0_v[r, pl.ds(j * 16, 16)] = jnp.zeros((16,), jnp.float32)
            return carry
        lax.fori_loop(0, C, zrow, 0)
        for z in range(ROWS_PER_TILE // C):
            pltpu.sync_copy(rows0_v, acc_sh.at[pl.ds(s * ROWS_PER_TILE + z * C, C)])
        plsc.subcore_barrier()

        stage_wait(0, 0)
        gather_start(0, 0)
        stage_wait(1, 1)
        gather_start(1, 1)

        def step(kk, carry):
            for b in range(4):
                k = kk * 4 + b
                rb = b % 2
                sb2 = (b + 2) % 4
                gather_wait(b, rb)

                def group(g, carry2):
                    wv = plsc.bitcast(stage_v[b, 2, pl.ds(g * 16, 16)],
                                      jnp.float32)
                    for l in range(16):
                        splat = _lane_splat(wv, l)
                        e = g * 16 + l
                        for j in range(F // 16):
                            rows[rb][e, pl.ds(j * 16, 16)] = (
                                rows[rb][e, pl.ds(j * 16, 16)] * splat)
                    return carry2
                lax.fori_loop(0, C // 16, group, 0)

                pltpu.sync_copy(rows[rb], acc_sh.at[stage_v.at[b, 1]], add=True)

                @pl.when(k + 4 < NCHUNK)
                def _():
                    stage_start(k + 4, b)

                @pl.when(k + 2 < NCHUNK)
                def _():
                    stage_wait(k + 2, sb2)
                    gather_start(sb2, rb)
            return carry
        lax.fori_loop(0, NCHUNK // 4, step, 0)

        plsc.subcore_barrier()
        for z in range(ROWS_PER_TILE // C):
            r0 = s * ROWS_PER_TILE + z * C
            pltpu.sync_copy(acc_sh.at[pl.ds(r0, C)], out_hbm.at[c, pl.ds(r0, C)])

    return agg


_agg_hid = _make_agg(NHID)
_agg_cls = _make_agg(NCLASS)


def _pad_edges(ei, ew):
    src = jnp.pad(ei[0], (0, E_PAD - N_EDGES)).reshape(NW, NCHUNK, 1, C)
    dst = jnp.pad(ei[1], (0, E_PAD - N_EDGES)).reshape(NW, NCHUNK, 1, C)
    w = lax.bitcast_convert_type(
        jnp.pad(ew, (0, E_PAD - N_EDGES)), jnp.int32).reshape(NW, NCHUNK, 1, C)
    return jnp.concatenate([src, dst, w], axis=2)


def kernel(x, edge_index1, edge_index2, edge_weight1, edge_weight2, W1, W2):
    xp = jnp.pad(x, ((0, N_PAD - N_NODES), (0, 0)))
    e1 = _pad_edges(edge_index1, edge_weight1)
    e2 = _pad_edges(edge_index2, edge_weight2)

    h1 = _matmul(xp, W1)
    p1 = _agg_hid(h1, e1)
    h2 = _relu_sum_matmul(p1, W2)
    p2 = _agg_cls(h2, e2)
    out = _sum2(p2)
    return out[:N_NODES]


# speedup vs baseline: 2.1268x; 2.1268x over previous
import functools

import jax
import jax.numpy as jnp
from jax import lax
from jax.experimental import pallas as pl
from jax.experimental.pallas import tpu as pltpu
from jax.experimental.pallas import tpu_sc as plsc

N_NODES = 10000
N_EDGES = 320000
NFEAT = 128
NHID = 128
NCLASS = 64

NW = 32
N_PAD = 10240
E_PAD = 327680
EPT = E_PAD // NW
C = 128
NCHUNK = EPT // C
ROWS_PER_TILE = N_PAD // 16


def _mm_body(x_ref, w_ref, o_ref):
    o_ref[...] = jnp.dot(x_ref[...], w_ref[...],
                         preferred_element_type=jnp.float32)


def _matmul(x, w, bm=1024):
    m, k = x.shape
    n = w.shape[1]
    return pl.pallas_call(
        _mm_body,
        grid=(m // bm,),
        in_specs=[pl.BlockSpec((bm, k), lambda i: (i, 0)),
                  pl.BlockSpec((k, n), lambda i: (0, 0))],
        out_specs=pl.BlockSpec((bm, n), lambda i: (i, 0)),
        out_shape=jax.ShapeDtypeStruct((m, n), jnp.float32),
    )(x, w)


def _mm2_body(p_ref, w_ref, o_ref):
    h = jnp.maximum(p_ref[0] + p_ref[1], 0.0)
    o_ref[...] = jnp.dot(h, w_ref[...], preferred_element_type=jnp.float32)


def _relu_sum_matmul(p, w, bm=1024):
    _, m, k = p.shape
    n = w.shape[1]
    return pl.pallas_call(
        _mm2_body,
        grid=(m // bm,),
        in_specs=[pl.BlockSpec((2, bm, k), lambda i: (0, i, 0)),
                  pl.BlockSpec((k, n), lambda i: (0, 0))],
        out_specs=pl.BlockSpec((bm, n), lambda i: (i, 0)),
        out_shape=jax.ShapeDtypeStruct((m, n), jnp.float32),
    )(p, w)


def _sum2_body(p_ref, o_ref):
    o_ref[...] = p_ref[0] + p_ref[1]


def _sum2(p, bm=1024):
    _, m, n = p.shape
    return pl.pallas_call(
        _sum2_body,
        grid=(m // bm,),
        in_specs=[pl.BlockSpec((2, bm, n), lambda i: (0, i, 0))],
        out_specs=pl.BlockSpec((bm, n), lambda i: (i, 0)),
        out_shape=jax.ShapeDtypeStruct((m, n), jnp.float32),
    )(p)


_GATHER_DNUMS = lax.GatherDimensionNumbers(
    offset_dims=(), collapsed_slice_dims=(0,), start_index_map=(0,))


def _lane_splat(vec, lane):
    idx = jnp.full((16, 1), lane, jnp.int32)
    return lax.gather(vec, idx, _GATHER_DNUMS, slice_sizes=(1,),
                      mode=lax.GatherScatterMode.PROMISE_IN_BOUNDS)


def _make_agg(F):
    mesh = plsc.VectorSubcoreMesh(core_axis_name="c", subcore_axis_name="s")

    @functools.partial(
        pl.kernel,
        out_type=jax.ShapeDtypeStruct((2, N_PAD, F), jnp.float32),
        mesh=mesh,
        compiler_params=pltpu.CompilerParams(use_tc_tiling_on_sc=False,
                                             needs_layout_passes=False),
        scratch_types=[
            pltpu.VMEM_SHARED((N_PAD, F), jnp.float32),
            pltpu.VMEM((4, 3, C), jnp.int32),
            pltpu.VMEM((C, F), jnp.float32),
            pltpu.VMEM((C, F), jnp.float32),
            pltpu.SemaphoreType.DMA,
            pltpu.SemaphoreType.DMA,
            pltpu.SemaphoreType.DMA,
            pltpu.SemaphoreType.DMA,
            pltpu.SemaphoreType.DMA,
            pltpu.SemaphoreType.DMA,
        ],
    )
    def agg(h_hbm, edges_hbm, out_hbm,
            acc_sh, stage_v, rows0_v, rows1_v,
            gsem0, gsem1, ssem0, ssem1, ssem2, ssem3):
        c = lax.axis_index("c")
        s = lax.axis_index("s")
        wid = s * 2 + c
        rows = (rows0_v, rows1_v)
        gsems = (gsem0, gsem1)
        ssems = (ssem0, ssem1, ssem2, ssem3)

        def stage_start(k, sb):
            pltpu.async_copy(edges_hbm.at[wid, k], stage_v.at[sb], ssems[sb])

        def stage_wait(k, sb):
            pltpu.make_async_copy(edges_hbm.at[wid, k], stage_v.at[sb],
                                  ssems[sb]).wait()

        def gather_start(sb, rb):
            pltpu.async_copy(h_hbm.at[stage_v.at[sb, 0]], rows[rb], gsems[rb])

        def gather_wait(sb, rb):
            pltpu.make_async_copy(h_hbm.at[stage_v.at[sb, 0]], rows[rb],
                                  gsems[rb]).wait()

        for sb in range(4):
            stage_start(sb, sb)

        def zrow(r, carry):
            for j in range(F // 16):
                rows0_v[r, pl.ds(j * 16, 16)] = jnp.zeros((16,), jnp.float32)
            return carry
        lax.fori_loop(0, C, zrow, 0)
        for z in range(ROWS_PER_TILE // C):
            pltpu.sync_copy(rows0_v, acc_sh.at[pl.ds(s * ROWS_PER_TILE + z * C, C)])
        plsc.subcore_barrier()

        stage_wait(0, 0)
        gather_start(0, 0)
        stage_wait(1, 1)
        gather_start(1, 1)

        def step(kk, carry):
            for b in range(4):
                k = kk * 4 + b
                rb = b % 2
                sb2 = (b + 2) % 4
                gather_wait(b, rb)

                def group(g, carry2):
                    wv = plsc.bitcast(stage_v[b, 2, pl.ds(g * 16, 16)],
                                      jnp.float32)
                    for l in range(16):
                        splat = _lane_splat(wv, l)
                        e = g * 16 + l
                        for j in range(F // 16):
                            rows[rb][e, pl.ds(j * 16, 16)] = (
                                rows[rb][e, pl.ds(j * 16, 16)] * splat)
                    return carry2
                lax.fori_loop(0, C // 16, group, 0)

                pltpu.sync_copy(rows[rb], acc_sh.at[stage_v.at[b, 1]], add=True)

                @pl.when(k + 4 < NCHUNK)
                def _():
                    stage_start(k + 4, b)

                @pl.when(k + 2 < NCHUNK)
                def _():
                    stage_wait(k + 2, sb2)
                    gather_start(sb2, rb)
            return carry
        lax.fori_loop(0, NCHUNK // 4, step, 0)

        plsc.subcore_barrier()
        for z in range(ROWS_PER_TILE // C):
            r0 = s * ROWS_PER_TILE + z * C
            pltpu.sync_copy(acc_sh.at[pl.ds(r0, C)], out_hbm.at[c, pl.ds(r0, C)])

    return agg


_agg_hid = _make_agg(NHID)
_agg_cls = _make_agg(NCLASS)


def _pad_edges(ei, ew):
    npad = E_PAD - N_EDGES
    spread = jnp.arange(npad, dtype=jnp.int32) % N_PAD
    src = jnp.concatenate([ei[0], spread]).reshape(NW, NCHUNK, 1, C)
    dst = jnp.concatenate([ei[1], spread]).reshape(NW, NCHUNK, 1, C)
    w = lax.bitcast_convert_type(
        jnp.pad(ew, (0, npad)), jnp.int32).reshape(NW, NCHUNK, 1, C)
    return jnp.concatenate([src, dst, w], axis=2)


def kernel(x, edge_index1, edge_index2, edge_weight1, edge_weight2, W1, W2):
    xp = jnp.pad(x, ((0, N_PAD - N_NODES), (0, 0)))
    e1 = _pad_edges(edge_index1, edge_weight1)
    e2 = _pad_edges(edge_index2, edge_weight2)

    h1 = _matmul(xp, W1)
    p1 = _agg_hid(h1, e1)
    h2 = _relu_sum_matmul(p1, W2)
    p2 = _agg_cls(h2, e2)
    out = _sum2(p2)
    return out[:N_NODES]
